# baseline (speedup 1.0000x reference)
import functools

import jax
import jax.numpy as jnp
from jax import lax
from jax.experimental import pallas as pl
from jax.experimental.pallas import tpu as pltpu
from jax.experimental.pallas import tpu_sc as plsc

B, N, NSYM = 16384, 200, 8
L = 16
NC, NS = 2, 16
NW = NC * NS
MOLS_PER_W = B // NW
MB = 128
NCHUNK = MOLS_PER_W // MB
NG = MB // L


def _sc_body(idx_hbm, c_hbm, w_hbm, out_hbm, wtab, ibuf, cbuf,
             si0, si1, sc0, sc1, so0, so1):
    wid = lax.axis_index("s") * NC + lax.axis_index("c")
    m0 = wid * MOLS_PER_W
    pltpu.sync_copy(w_hbm, wtab.at[pl.ds(0, NSYM)])
    wreg = wtab[...]

    zf = jnp.zeros((L,), jnp.float32)
    sem_i = (si0, si1)
    sem_c = (sc0, sc1)
    sem_o = (so0, so1)

    def col(k):
        return pl.ds(m0 + k * MB, MB)

    def start_in(k):
        b = k % 2
        return (
            pltpu.async_copy(idx_hbm.at[:, col(k)], ibuf.at[b], sem_i[b]),
            pltpu.async_copy(c_hbm.at[:, col(k)], cbuf.at[b], sem_c[b]),
        )

    pending_in = {0: start_in(0)}
    pending_out = {}
    for k in range(NCHUNK):
        b = k % 2
        if k + 1 < NCHUNK:
            if k - 1 in pending_out:
                pending_out.pop(k - 1).wait()
            pending_in[k + 1] = start_in(k + 1)
        cp_i, cp_c = pending_in.pop(k)
        cp_i.wait()
        cp_c.wait()

        def sum_body(t, accs, b=b):
            n = t * 2
            accs = list(accs)
            for g in range(NG):
                c0 = cbuf[b, n, pl.ds(L * g, L)]
                c1 = cbuf[b, n + 1, pl.ds(L * g, L)]
                i0 = plsc.bitcast(ibuf[b, n, pl.ds(L * g, L)], jnp.int32)
                i1 = plsc.bitcast(ibuf[b, n + 1, pl.ds(L * g, L)], jnp.int32)
                w0 = jnp.take_along_axis(wreg, i0, axis=0)
                w1 = jnp.take_along_axis(wreg, i1, axis=0)
                ac, aw = accs[g]
                accs[g] = (ac + (c0 + c1), aw + (w0 + w1))
            return tuple(accs)

        accs = lax.fori_loop(0, N // 2, sum_body,
                             tuple((zf, zf) for _ in range(NG)))
        scales = [(0.0 - ac) / aw for ac, aw in accs]

        def out_body(t, carry, b=b, scales=scales):
            n = t * 2
            for dn in range(2):
                for g in range(NG):
                    c = cbuf[b, n + dn, pl.ds(L * g, L)]
                    ix = plsc.bitcast(ibuf[b, n + dn, pl.ds(L * g, L)], jnp.int32)
                    w = jnp.take_along_axis(wreg, ix, axis=0)
                    ibuf[b, n + dn, pl.ds(L * g, L)] = c + scales[g] * w
            return carry

        lax.fori_loop(0, N // 2, out_body, 0)
        pending_out[k] = pltpu.async_copy(
            ibuf.at[b], out_hbm.at[:, col(k)], sem_o[b]
        )

    for k in sorted(pending_out):
        pending_out[k].wait()


def kernel(element_idxs, raw_charges, weights):
    mesh = plsc.VectorSubcoreMesh(core_axis_name="c", subcore_axis_name="s")
    f = pl.kernel(
        _sc_body,
        mesh=mesh,
        compiler_params=pltpu.CompilerParams(needs_layout_passes=False),
        out_type=jax.ShapeDtypeStruct((N, B), jnp.float32),
        scratch_types=[
            pltpu.VMEM((L,), jnp.float32),
            pltpu.VMEM((2, N, MB), jnp.float32),
            pltpu.VMEM((2, N, MB), jnp.float32),
            pltpu.SemaphoreType.DMA,
            pltpu.SemaphoreType.DMA,
            pltpu.SemaphoreType.DMA,
            pltpu.SemaphoreType.DMA,
            pltpu.SemaphoreType.DMA,
            pltpu.SemaphoreType.DMA,
        ],
    )
    out_t = f(element_idxs.T.view(jnp.float32), raw_charges.T, weights)
    return out_t.T


# speedup vs baseline: 1.0104x; 1.0104x over previous
import functools

import jax
import jax.numpy as jnp
from jax import lax
from jax.experimental import pallas as pl
from jax.experimental.pallas import tpu as pltpu
from jax.experimental.pallas import tpu_sc as plsc

B, N, NSYM = 16384, 200, 8
L = 16
NC, NS = 2, 16
NW = NC * NS
MOLS_PER_W = B // NW
MB = 128
NCHUNK = MOLS_PER_W // MB
NG = MB // L


def _sc_body(idx_hbm, c_hbm, w_hbm, out_hbm, wtab, ibuf, cbuf,
             si0, si1, sc0, sc1, so0, so1):
    wid = lax.axis_index("s") * NC + lax.axis_index("c")
    m0 = wid * MOLS_PER_W
    pltpu.sync_copy(w_hbm, wtab.at[pl.ds(0, NSYM)])
    wreg = wtab[...]

    zf = jnp.zeros((L,), jnp.float32)
    sem_i = (si0, si1)
    sem_c = (sc0, sc1)
    sem_o = (so0, so1)

    def col(k):
        return pl.ds(m0 + k * MB, MB)

    def start_in(k):
        b = k % 2
        return (
            pltpu.async_copy(idx_hbm.at[:, col(k)], ibuf.at[b], sem_i[b]),
            pltpu.async_copy(c_hbm.at[:, col(k)], cbuf.at[b], sem_c[b]),
        )

    pending_in = {0: start_in(0)}
    pending_out = {}
    for k in range(NCHUNK):
        b = k % 2
        if k + 1 < NCHUNK:
            if k - 1 in pending_out:
                pending_out.pop(k - 1).wait()
            pending_in[k + 1] = start_in(k + 1)
        cp_i, cp_c = pending_in.pop(k)
        cp_i.wait()
        cp_c.wait()

        def sum_body(n, accs, b=b):
            accs = list(accs)
            for g in range(NG):
                c = cbuf[b, n, pl.ds(L * g, L)]
                ix = plsc.bitcast(ibuf[b, n, pl.ds(L * g, L)], jnp.int32)
                w = jnp.take_along_axis(wreg, ix, axis=0)
                ac, aw = accs[g]
                accs[g] = (ac + c, aw + w)
            return tuple(accs)

        accs = lax.fori_loop(0, N, sum_body, tuple((zf, zf) for _ in range(NG)))
        scales = [(0.0 - ac) / aw for ac, aw in accs]

        def out_body(n, carry, b=b, scales=scales):
            for g in range(NG):
                c = cbuf[b, n, pl.ds(L * g, L)]
                ix = plsc.bitcast(ibuf[b, n, pl.ds(L * g, L)], jnp.int32)
                w = jnp.take_along_axis(wreg, ix, axis=0)
                ibuf[b, n, pl.ds(L * g, L)] = c + scales[g] * w
            return carry

        lax.fori_loop(0, N, out_body, 0)
        pending_out[k] = pltpu.async_copy(
            ibuf.at[b], out_hbm.at[:, col(k)], sem_o[b]
        )

    for k in sorted(pending_out):
        pending_out[k].wait()


def kernel(element_idxs, raw_charges, weights):
    mesh = plsc.VectorSubcoreMesh(core_axis_name="c", subcore_axis_name="s")
    f = pl.kernel(
        _sc_body,
        mesh=mesh,
        compiler_params=pltpu.CompilerParams(needs_layout_passes=False),
        out_type=jax.ShapeDtypeStruct((N, B), jnp.float32),
        scratch_types=[
            pltpu.VMEM((L,), jnp.float32),
            pltpu.VMEM((2, N, MB), jnp.float32),
            pltpu.VMEM((2, N, MB), jnp.float32),
            pltpu.SemaphoreType.DMA,
            pltpu.SemaphoreType.DMA,
            pltpu.SemaphoreType.DMA,
            pltpu.SemaphoreType.DMA,
            pltpu.SemaphoreType.DMA,
            pltpu.SemaphoreType.DMA,
        ],
    )
    out_t = f(element_idxs.T.view(jnp.float32), raw_charges.T, weights)
    return out_t.T


# speedup vs baseline: 1.0302x; 1.0196x over previous
import functools

import jax
import jax.numpy as jnp
from jax import lax
from jax.experimental import pallas as pl
from jax.experimental.pallas import tpu as pltpu
from jax.experimental.pallas import tpu_sc as plsc

B, N, NSYM = 16384, 200, 8
L = 16
NC, NS = 2, 16
NW = NC * NS
MOLS_PER_W = B // NW
MB = 128
NCHUNK = MOLS_PER_W // MB
NG = MB // L


def _sc_body(idx_hbm, c_hbm, w_hbm, out_hbm, wtab, ibuf, cbuf,
             si0, si1, sc0, sc1, so0, so1):
    wid = lax.axis_index("s") * NC + lax.axis_index("c")
    m0 = wid * MOLS_PER_W

    zf = jnp.zeros((L,), jnp.float32)
    sem_i = (si0, si1)
    sem_c = (sc0, sc1)
    sem_o = (so0, so1)

    def col(k):
        return pl.ds(m0 + k * MB, MB)

    def start_in(k):
        b = k % 2
        return (
            pltpu.async_copy(idx_hbm.at[:, col(k)], ibuf.at[b], sem_i[b]),
            pltpu.async_copy(c_hbm.at[:, col(k)], cbuf.at[b], sem_c[b]),
        )

    pending_in = {0: start_in(0)}
    pltpu.sync_copy(w_hbm, wtab.at[pl.ds(0, NSYM)])
    wreg = wtab[...]
    pending_out = {}
    for k in range(NCHUNK):
        b = k % 2
        if k + 1 < NCHUNK:
            if k - 1 in pending_out:
                pending_out.pop(k - 1).wait()
            pending_in[k + 1] = start_in(k + 1)
        cp_i, cp_c = pending_in.pop(k)
        cp_i.wait()
        cp_c.wait()

        def sum_body(n, accs, b=b):
            accs = list(accs)
            for g in range(NG):
                c = cbuf[b, n, pl.ds(L * g, L)]
                ix = plsc.bitcast(ibuf[b, n, pl.ds(L * g, L)], jnp.int32)
                w = jnp.take_along_axis(wreg, ix, axis=0)
                ac, aw = accs[g]
                accs[g] = (ac + c, aw + w)
            return tuple(accs)

        accs = lax.fori_loop(0, N, sum_body, tuple((zf, zf) for _ in range(NG)))
        scales = [(0.0 - ac) / aw for ac, aw in accs]

        def out_body(n, carry, b=b, scales=scales):
            for g in range(NG):
                c = cbuf[b, n, pl.ds(L * g, L)]
                ix = plsc.bitcast(ibuf[b, n, pl.ds(L * g, L)], jnp.int32)
                w = jnp.take_along_axis(wreg, ix, axis=0)
                ibuf[b, n, pl.ds(L * g, L)] = c + scales[g] * w
            return carry

        lax.fori_loop(0, N, out_body, 0)
        pending_out[k] = pltpu.async_copy(
            ibuf.at[b], out_hbm.at[:, col(k)], sem_o[b]
        )

    for k in sorted(pending_out):
        pending_out[k].wait()


def kernel(element_idxs, raw_charges, weights):
    mesh = plsc.VectorSubcoreMesh(core_axis_name="c", subcore_axis_name="s")
    f = pl.kernel(
        _sc_body,
        mesh=mesh,
        compiler_params=pltpu.CompilerParams(needs_layout_passes=False),
        out_type=jax.ShapeDtypeStruct((N, B), jnp.float32),
        scratch_types=[
            pltpu.VMEM((L,), jnp.float32),
            pltpu.VMEM((2, N, MB), jnp.float32),
            pltpu.VMEM((2, N, MB), jnp.float32),
            pltpu.SemaphoreType.DMA,
            pltpu.SemaphoreType.DMA,
            pltpu.SemaphoreType.DMA,
            pltpu.SemaphoreType.DMA,
            pltpu.SemaphoreType.DMA,
            pltpu.SemaphoreType.DMA,
        ],
    )
    out_t = f(element_idxs.T.view(jnp.float32), raw_charges.T, weights)
    return out_t.T
